# baseline (speedup 1.0000x reference)
import jax
import jax.numpy as jnp
from jax.experimental import pallas as pl
from jax.experimental.pallas import tpu as pltpu


def kernel(x_nchw, weight_oihw, bias):
    N, C, H, W = x_nchw.shape
    Ho, Wo = H // 2, W // 2
    HW = H * W
    M = Ho * Wo

    xi = jax.lax.bitcast_convert_type(
        x_nchw.reshape(N, C, HW // 2, 2).astype(jnp.bfloat16), jnp.int32
    )

    wt = (
        jnp.transpose(weight_oihw, (2, 3, 0, 1))
        .reshape(9, C, C)
        .astype(jnp.bfloat16)
    )
    b2 = bias.reshape(C, 1)
    eye2 = jnp.zeros((2 * C, 2 * C), jnp.bfloat16)
    eye2 = eye2.at[jnp.arange(2 * C), jnp.arange(2 * C)].set(jnp.bfloat16(1))
    sel = eye2.reshape(C, 2, 2 * C)
    i_even = sel[:, 0, :].T
    i_odd = sel[:, 1, :].T

    def body(x_ref, w_ref, b_ref, ie_ref, io_ref, o_ref):
        xb2 = pltpu.bitcast(x_ref[0], jnp.bfloat16)

        def tr(i_ref_):
            t = jax.lax.dot_general(
                xb2,
                i_ref_[:, :],
                dimension_numbers=(((0,), (0,)), ((), ())),
                preferred_element_type=jnp.float32,
            ).astype(jnp.bfloat16)
            return t.reshape(Ho, 2, Wo, C)

        xe = tr(ie_ref)
        xo = tr(io_ref)

        t11 = xe[:, 0]
        t21 = xe[:, 1]
        t12 = xo[:, 0]
        t22 = xo[:, 1]

        zrow = jnp.zeros((1, Wo, C), jnp.bfloat16)
        zcol = jnp.zeros((Ho, 1, C), jnp.bfloat16)

        def rshift(t):
            return jnp.concatenate([zrow, t[:-1]], axis=0)

        def cshift(t):
            return jnp.concatenate([zcol, t[:, :-1, :]], axis=1)

        taps = (
            (0, rshift(cshift(t22))), (1, rshift(t21)), (2, rshift(t22)),
            (3, cshift(t12)), (4, t11), (5, t12),
            (6, cshift(t22)), (7, t21), (8, t22),
        )
        dn = (((1,), (1,)), ((), ()))
        acc = jnp.zeros((C, M), jnp.float32)
        for t, tap in taps:
            acc = acc + jax.lax.dot_general(
                w_ref[t],
                tap.reshape(M, C),
                dimension_numbers=dn,
                preferred_element_type=jnp.float32,
            )
        o_ref[0] = acc + b_ref[:, :]

    out = pl.pallas_call(
        body,
        out_shape=jax.ShapeDtypeStruct((N, C, M), jnp.float32),
        grid=(N,),
        in_specs=[
            pl.BlockSpec((1, C, HW // 2), lambda n: (n, 0, 0)),
            pl.BlockSpec((9, C, C), lambda n: (0, 0, 0)),
            pl.BlockSpec((C, 1), lambda n: (0, 0)),
            pl.BlockSpec((2 * C, C), lambda n: (0, 0)),
            pl.BlockSpec((2 * C, C), lambda n: (0, 0)),
        ],
        out_specs=pl.BlockSpec((1, C, M), lambda n: (n, 0, 0)),
        compiler_params=pltpu.CompilerParams(
            dimension_semantics=("arbitrary",),
            vmem_limit_bytes=100 * 1024 * 1024,
        ),
    )(xi, wt, b2, i_even, i_odd)

    return out.reshape(N, C, Ho, Wo)


# speedup vs baseline: 1.0045x; 1.0045x over previous
import jax
import jax.numpy as jnp
from jax.experimental import pallas as pl
from jax.experimental.pallas import tpu as pltpu


def kernel(x_nchw, weight_oihw, bias):
    N, C, H, W = x_nchw.shape
    Ho, Wo = H // 2, W // 2
    HW = H * W
    M = Ho * Wo

    xi = jax.lax.bitcast_convert_type(
        x_nchw.reshape(N, C, HW // 2, 2).astype(jnp.bfloat16), jnp.int32
    )

    wt = (
        jnp.transpose(weight_oihw, (2, 3, 0, 1))
        .reshape(9, C, C)
        .astype(jnp.bfloat16)
    )
    b2 = bias.reshape(C, 1)
    r_ix = jax.lax.broadcasted_iota(jnp.int32, (2 * C, C), 0)
    c_ix = jax.lax.broadcasted_iota(jnp.int32, (2 * C, C), 1)
    i_even = (r_ix == 2 * c_ix).astype(jnp.bfloat16)
    i_odd = (r_ix == 2 * c_ix + 1).astype(jnp.bfloat16)

    def body(x_ref, w_ref, b_ref, ie_ref, io_ref, o_ref):
        xb2 = pltpu.bitcast(x_ref[0], jnp.bfloat16)

        def tr(i_ref_):
            t = jax.lax.dot_general(
                xb2,
                i_ref_[:, :],
                dimension_numbers=(((0,), (0,)), ((), ())),
                preferred_element_type=jnp.float32,
            ).astype(jnp.bfloat16)
            return t.reshape(Ho, 2, Wo, C)

        xe = tr(ie_ref)
        xo = tr(io_ref)

        t11 = xe[:, 0]
        t21 = xe[:, 1]
        t12 = xo[:, 0]
        t22 = xo[:, 1]

        zrow = jnp.zeros((1, Wo, C), jnp.bfloat16)
        zcol = jnp.zeros((Ho, 1, C), jnp.bfloat16)

        def rshift(t):
            return jnp.concatenate([zrow, t[:-1]], axis=0)

        def cshift(t):
            return jnp.concatenate([zcol, t[:, :-1, :]], axis=1)

        taps = (
            (0, rshift(cshift(t22))), (1, rshift(t21)), (2, rshift(t22)),
            (3, cshift(t12)), (4, t11), (5, t12),
            (6, cshift(t22)), (7, t21), (8, t22),
        )
        dn = (((1,), (1,)), ((), ()))
        acc = jnp.zeros((C, M), jnp.float32)
        for t, tap in taps:
            acc = acc + jax.lax.dot_general(
                w_ref[t],
                tap.reshape(M, C),
                dimension_numbers=dn,
                preferred_element_type=jnp.float32,
            )
        o_ref[0] = acc + b_ref[:, :]

    out = pl.pallas_call(
        body,
        out_shape=jax.ShapeDtypeStruct((N, C, M), jnp.float32),
        grid=(N,),
        in_specs=[
            pl.BlockSpec((1, C, HW // 2), lambda n: (n, 0, 0)),
            pl.BlockSpec((9, C, C), lambda n: (0, 0, 0)),
            pl.BlockSpec((C, 1), lambda n: (0, 0)),
            pl.BlockSpec((2 * C, C), lambda n: (0, 0)),
            pl.BlockSpec((2 * C, C), lambda n: (0, 0)),
        ],
        out_specs=pl.BlockSpec((1, C, M), lambda n: (n, 0, 0)),
        compiler_params=pltpu.CompilerParams(
            dimension_semantics=("arbitrary",),
            vmem_limit_bytes=100 * 1024 * 1024,
        ),
    )(xi, wt, b2, i_even, i_odd)

    return out.reshape(N, C, Ho, Wo)


# speedup vs baseline: 2.8769x; 2.8639x over previous
import jax
import jax.numpy as jnp
from jax.experimental import pallas as pl
from jax.experimental.pallas import tpu as pltpu


def kernel(x_nchw, weight_oihw, bias):
    N, C, H, W = x_nchw.shape
    Ho, Wo = H // 2, W // 2
    HW = H * W
    M = Ho * Wo

    x2 = x_nchw.reshape(N, C, HW)

    wt = (
        jnp.transpose(weight_oihw, (2, 3, 0, 1))
        .reshape(9, C, C)
        .astype(jnp.bfloat16)
    )
    b2 = bias.reshape(C, 1)
    ident = jnp.eye(C, dtype=jnp.bfloat16)

    def body(x_ref, w_ref, b_ref, i_ref, o_ref, sa_ref, sb_ref):
        n = pl.program_id(0)
        p = jax.lax.rem(n, 2)

        @pl.when(n < N // 2)
        def _transpose():
            for u in range(2):
                xb = x_ref[u].astype(jnp.bfloat16)
                xt = jax.lax.dot_general(
                    xb,
                    i_ref[:, :],
                    dimension_numbers=(((0,), (0,)), ((), ())),
                    preferred_element_type=jnp.float32,
                )
                sa_ref[p, u] = xt[:, :128].reshape(H, W, 128)
                sb_ref[p, u] = xt[:, 128:].reshape(H, W, 128)

        @pl.when(n >= 1)
        def _conv():
          q = 1 - p
          for u in range(2):
            def phase(rp, cp):
                lo = sa_ref.at[q].at[u][pl.ds(rp, Ho, 2), pl.ds(cp, Wo, 2), :]
                hi = sb_ref.at[q].at[u][pl.ds(rp, Ho, 2), pl.ds(cp, Wo, 2), :]
                return jnp.concatenate([lo, hi], axis=-1).astype(jnp.bfloat16)

            t11 = phase(0, 0)
            t12 = phase(0, 1)
            t21 = phase(1, 0)
            t22 = phase(1, 1)

            zrow = jnp.zeros((1, Wo, C), jnp.bfloat16)
            zcol = jnp.zeros((Ho, 1, C), jnp.bfloat16)

            def rshift(t):
                return jnp.concatenate([zrow, t[:-1]], axis=0)

            def cshift(t):
                return jnp.concatenate([zcol, t[:, :-1, :]], axis=1)

            taps = (
                (0, rshift(cshift(t22))), (1, rshift(t21)), (2, rshift(t22)),
                (3, cshift(t12)), (4, t11), (5, t12),
                (6, cshift(t22)), (7, t21), (8, t22),
            )
            dn = (((1,), (1,)), ((), ()))
            acc = jnp.zeros((C, M), jnp.float32)
            for t, tap in taps:
                acc = acc + jax.lax.dot_general(
                    w_ref[t],
                    tap.reshape(M, C),
                    dimension_numbers=dn,
                    preferred_element_type=jnp.float32,
                )
            o_ref[u] = acc + b_ref[:, :]

    out = pl.pallas_call(
        body,
        out_shape=jax.ShapeDtypeStruct((N, C, M), jnp.float32),
        grid=(N // 2 + 1,),
        in_specs=[
            pl.BlockSpec((2, C, HW), lambda n: (jnp.minimum(n, N // 2 - 1), 0, 0)),
            pl.BlockSpec((9, C, C), lambda n: (0, 0, 0)),
            pl.BlockSpec((C, 1), lambda n: (0, 0)),
            pl.BlockSpec((C, C), lambda n: (0, 0)),
        ],
        out_specs=pl.BlockSpec(
            (2, C, M), lambda n: (jnp.maximum(n - 1, 0), 0, 0)
        ),
        scratch_shapes=[
            pltpu.VMEM((2, 2, H, W, 128), jnp.float32),
            pltpu.VMEM((2, 2, H, W, 128), jnp.float32),
        ],
        compiler_params=pltpu.CompilerParams(
            dimension_semantics=("arbitrary",),
            vmem_limit_bytes=100 * 1024 * 1024,
        ),
    )(x2, wt, b2, ident)

    return out.reshape(N, C, Ho, Wo)
